# initial kernel scaffold (unmeasured)
import jax
import jax.numpy as jnp
from jax import lax
from jax.experimental import pallas as pl
from jax.experimental.pallas import tpu as pltpu

N_DEV = 4
M, N = 4096, 8192
MB = M // N_DEV
NC = 2048
N_CHUNKS = N // NC
N_HOPS = 2 * (N_DEV - 1)


def _ar_body(p_ref, y_ref, amax_ref, sbuf, pbuf, comm, load_sem, store_sem,
             send_sems, recv_sems, credit_sem):
    me = lax.axis_index("i")
    left = lax.rem(me + N_DEV - 1, N_DEV)
    right = lax.rem(me + 1, N_DEV)

    barrier = pltpu.get_barrier_semaphore()
    for nbr in (left, right):
        pl.semaphore_signal(barrier, inc=1, device_id=(nbr,),
                            device_id_type=pl.DeviceIdType.MESH)
    pl.semaphore_wait(barrier, 2)

    amax_ref[0, 0] = 0.0

    for c in range(N_CHUNKS):
        col = c * NC
        ld = pltpu.make_async_copy(
            p_ref.at[pl.ds(me * MB, MB), pl.ds(col, NC)], sbuf, load_sem)
        ld.start()
        ld.wait()
        for h in range(N_HOPS):
            slot = h % 2
            if c * N_HOPS + h >= 2:
                pl.semaphore_wait(credit_sem, 1)
            rdma = pltpu.make_async_remote_copy(
                src_ref=sbuf,
                dst_ref=comm.at[slot],
                send_sem=send_sems.at[slot],
                recv_sem=recv_sems.at[slot],
                device_id=(right,),
                device_id_type=pl.DeviceIdType.MESH,
            )
            rdma.start()
            rdma.wait()
            rb = lax.rem(me + 7 - h, N_DEV)
            if h < N_DEV - 1:
                ld = pltpu.make_async_copy(
                    p_ref.at[pl.ds(rb * MB, MB), pl.ds(col, NC)], pbuf,
                    load_sem)
                ld.start()
                ld.wait()
                sbuf[...] = comm[slot] + pbuf[...]
            else:
                sbuf[...] = comm[slot]
            pl.semaphore_signal(credit_sem, inc=1, device_id=(left,),
                                device_id_type=pl.DeviceIdType.MESH)
            if h >= N_DEV - 2:
                st = pltpu.make_async_copy(
                    sbuf, y_ref.at[pl.ds(rb * MB, MB), pl.ds(col, NC)],
                    store_sem)
                st.start()
                st.wait()
                amax_ref[0, 0] = jnp.maximum(
                    amax_ref[0, 0], jnp.max(sbuf[...]).astype(jnp.float32))

    pl.semaphore_wait(credit_sem, 2)


def _all_reduce(partial):
    return pl.pallas_call(
        _ar_body,
        out_shape=(
            jax.ShapeDtypeStruct((M, N), jnp.bfloat16),
            jax.ShapeDtypeStruct((1, 1), jnp.float32),
        ),
        in_specs=[pl.BlockSpec(memory_space=pltpu.ANY)],
        out_specs=(
            pl.BlockSpec(memory_space=pltpu.ANY),
            pl.BlockSpec(memory_space=pltpu.SMEM),
        ),
        scratch_shapes=[
            pltpu.VMEM((MB, NC), jnp.bfloat16),
            pltpu.VMEM((MB, NC), jnp.bfloat16),
            pltpu.VMEM((2, MB, NC), jnp.bfloat16),
            pltpu.SemaphoreType.DMA,
            pltpu.SemaphoreType.DMA,
            pltpu.SemaphoreType.DMA((2,)),
            pltpu.SemaphoreType.DMA((2,)),
            pltpu.SemaphoreType.REGULAR,
        ],
        compiler_params=pltpu.CompilerParams(collective_id=0),
    )(partial)


def _epilogue(y, amax):
    TM = 256

    def body(amax_ref, y_ref, o_ref):
        yv = jnp.maximum(y_ref[...].astype(jnp.float32), 0.0)
        scale = amax_ref[0, 0] / 127.0
        q = jnp.clip(jnp.round(yv / scale), 0.0, 127.0)
        o_ref[...] = q * scale

    return pl.pallas_call(
        body,
        grid=(M // TM,),
        out_shape=jax.ShapeDtypeStruct((M, N), jnp.float32),
        in_specs=[
            pl.BlockSpec(memory_space=pltpu.SMEM),
            pl.BlockSpec((TM, N), lambda i: (i, 0)),
        ],
        out_specs=pl.BlockSpec((TM, N), lambda i: (i, 0)),
    )(amax, y)


def kernel(x, w_mat):
    xb = x.astype(jnp.bfloat16)
    wb = w_mat.astype(jnp.bfloat16)
    partial = jnp.dot(
        xb, wb, preferred_element_type=jnp.float32).astype(jnp.bfloat16)
    y, amax = _all_reduce(partial)
    return _epilogue(y, amax)


# baseline (device time: 1408193 ns/iter reference)
import jax
import jax.numpy as jnp
from jax import lax
from jax.experimental import pallas as pl
from jax.experimental.pallas import tpu as pltpu

N_DEV = 4
M, N = 4096, 8192
MB = M // N_DEV
NC = 2048
N_CHUNKS = N // NC
N_HOPS = 2 * (N_DEV - 1)


def _ar_body(p_ref, y_ref, amax_ref, sbuf, pbuf, comm, load_sem, store_sem,
             send_sems, recv_sems, credit_sem):
    me = lax.axis_index("i")
    left = lax.rem(me + N_DEV - 1, N_DEV)
    right = lax.rem(me + 1, N_DEV)

    barrier = pltpu.get_barrier_semaphore()
    for nbr in (left, right):
        pl.semaphore_signal(barrier, inc=1, device_id=(nbr,),
                            device_id_type=pl.DeviceIdType.MESH)
    pl.semaphore_wait(barrier, 2)

    amax_ref[0, 0] = 0.0

    for c in range(N_CHUNKS):
        col = c * NC
        ld = pltpu.make_async_copy(
            p_ref.at[pl.ds(me * MB, MB), pl.ds(col, NC)], sbuf, load_sem)
        ld.start()
        ld.wait()
        for h in range(N_HOPS):
            slot = h % 2
            if c * N_HOPS + h >= 2:
                pl.semaphore_wait(credit_sem, 1)
            rdma = pltpu.make_async_remote_copy(
                src_ref=sbuf,
                dst_ref=comm.at[slot],
                send_sem=send_sems.at[slot],
                recv_sem=recv_sems.at[slot],
                device_id=(right,),
                device_id_type=pl.DeviceIdType.MESH,
            )
            rdma.start()
            rdma.wait()
            rb = lax.rem(me + 7 - h, N_DEV)
            if h < N_DEV - 1:
                ld = pltpu.make_async_copy(
                    p_ref.at[pl.ds(rb * MB, MB), pl.ds(col, NC)], pbuf,
                    load_sem)
                ld.start()
                ld.wait()
                sbuf[...] = comm[slot] + pbuf[...]
            else:
                sbuf[...] = comm[slot]
            pl.semaphore_signal(credit_sem, inc=1, device_id=(left,),
                                device_id_type=pl.DeviceIdType.MESH)
            if h >= N_DEV - 2:
                st = pltpu.make_async_copy(
                    sbuf, y_ref.at[pl.ds(rb * MB, MB), pl.ds(col, NC)],
                    store_sem)
                st.start()
                st.wait()
                amax_ref[0, 0] = jnp.maximum(
                    amax_ref[0, 0], jnp.max(sbuf[...].astype(jnp.float32)))

    pl.semaphore_wait(credit_sem, 2)


def _all_reduce(partial):
    return pl.pallas_call(
        _ar_body,
        out_shape=(
            jax.ShapeDtypeStruct((M, N), jnp.bfloat16),
            jax.ShapeDtypeStruct((1, 1), jnp.float32),
        ),
        in_specs=[pl.BlockSpec(memory_space=pl.ANY)],
        out_specs=(
            pl.BlockSpec(memory_space=pl.ANY),
            pl.BlockSpec(memory_space=pltpu.MemorySpace.SMEM),
        ),
        scratch_shapes=[
            pltpu.VMEM((MB, NC), jnp.bfloat16),
            pltpu.VMEM((MB, NC), jnp.bfloat16),
            pltpu.VMEM((2, MB, NC), jnp.bfloat16),
            pltpu.SemaphoreType.DMA,
            pltpu.SemaphoreType.DMA,
            pltpu.SemaphoreType.DMA((2,)),
            pltpu.SemaphoreType.DMA((2,)),
            pltpu.SemaphoreType.REGULAR,
        ],
        compiler_params=pltpu.CompilerParams(collective_id=0),
    )(partial)


def _epilogue(y, amax):
    TM = 256

    def body(amax_ref, y_ref, o_ref):
        yv = jnp.maximum(y_ref[...].astype(jnp.float32), 0.0)
        scale = amax_ref[0, 0] / 127.0
        q = jnp.clip(jnp.round(yv / scale), 0.0, 127.0)
        o_ref[...] = q * scale

    return pl.pallas_call(
        body,
        grid=(M // TM,),
        out_shape=jax.ShapeDtypeStruct((M, N), jnp.float32),
        in_specs=[
            pl.BlockSpec(memory_space=pltpu.MemorySpace.SMEM),
            pl.BlockSpec((TM, N), lambda i: (i, 0)),
        ],
        out_specs=pl.BlockSpec((TM, N), lambda i: (i, 0)),
    )(amax, y)


def kernel(x, w_mat):
    xb = x.astype(jnp.bfloat16)
    wb = w_mat.astype(jnp.bfloat16)
    partial = jnp.dot(
        xb, wb, preferred_element_type=jnp.float32).astype(jnp.bfloat16)
    y, amax = _all_reduce(partial)
    return _epilogue(y, amax)


# device time: 791890 ns/iter; 1.7783x vs baseline; 1.7783x over previous
import jax
import jax.numpy as jnp
from jax import lax
from jax.experimental import pallas as pl
from jax.experimental.pallas import tpu as pltpu

N_DEV = 4
M, N = 4096, 8192
MB = M // N_DEV
NC = 2048
N_CHUNKS = N // NC
N_HOPS = 2 * (N_DEV - 1)


def _ar_body(p_ref, y_ref, amax_ref,
             sbuf_r, sbuf_l, pbuf_r, pbuf_l, comm_r, comm_l,
             load_sems, store_sems,
             send_r, recv_r, send_l, recv_l,
             credit_r, credit_l):
    me = lax.axis_index("i")
    left = lax.rem(me + N_DEV - 1, N_DEV)
    right = lax.rem(me + 1, N_DEV)

    barrier = pltpu.get_barrier_semaphore()
    for nbr in (left, right):
        pl.semaphore_signal(barrier, inc=1, device_id=(nbr,),
                            device_id_type=pl.DeviceIdType.MESH)
    pl.semaphore_wait(barrier, 2)

    amax_ref[0, 0] = 0.0

    n_pairs = N_CHUNKS // 2
    for p in range(n_pairs):
        col_r = p * NC
        col_l = (p + n_pairs) * NC
        ld0 = pltpu.make_async_copy(
            p_ref.at[pl.ds(me * MB, MB), pl.ds(col_r, NC)], sbuf_r,
            load_sems.at[0])
        ld1 = pltpu.make_async_copy(
            p_ref.at[pl.ds(me * MB, MB), pl.ds(col_l, NC)], sbuf_l,
            load_sems.at[1])
        ld0.start()
        ld1.start()
        ld0.wait()
        ld1.wait()
        for h in range(N_HOPS):
            slot = h % 2
            if p * N_HOPS + h >= 2:
                pl.semaphore_wait(credit_r, 1)
                pl.semaphore_wait(credit_l, 1)
            rd_r = pltpu.make_async_remote_copy(
                src_ref=sbuf_r, dst_ref=comm_r.at[slot],
                send_sem=send_r.at[slot], recv_sem=recv_r.at[slot],
                device_id=(right,), device_id_type=pl.DeviceIdType.MESH)
            rd_l = pltpu.make_async_remote_copy(
                src_ref=sbuf_l, dst_ref=comm_l.at[slot],
                send_sem=send_l.at[slot], recv_sem=recv_l.at[slot],
                device_id=(left,), device_id_type=pl.DeviceIdType.MESH)
            rd_r.start()
            rd_l.start()
            rb_r = lax.rem(me + 7 - h, N_DEV)
            rb_l = lax.rem(me + h + 1, N_DEV)
            if h < N_DEV - 1:
                ld0 = pltpu.make_async_copy(
                    p_ref.at[pl.ds(rb_r * MB, MB), pl.ds(col_r, NC)],
                    pbuf_r, load_sems.at[0])
                ld1 = pltpu.make_async_copy(
                    p_ref.at[pl.ds(rb_l * MB, MB), pl.ds(col_l, NC)],
                    pbuf_l, load_sems.at[1])
                ld0.start()
                ld1.start()
            rd_r.wait()
            rd_l.wait()
            if h < N_DEV - 1:
                ld0.wait()
                ld1.wait()
                sbuf_r[...] = comm_r[slot] + pbuf_r[...]
                sbuf_l[...] = comm_l[slot] + pbuf_l[...]
            else:
                sbuf_r[...] = comm_r[slot]
                sbuf_l[...] = comm_l[slot]
            pl.semaphore_signal(credit_r, inc=1, device_id=(left,),
                                device_id_type=pl.DeviceIdType.MESH)
            pl.semaphore_signal(credit_l, inc=1, device_id=(right,),
                                device_id_type=pl.DeviceIdType.MESH)
            if h >= N_DEV - 2:
                st0 = pltpu.make_async_copy(
                    sbuf_r, y_ref.at[pl.ds(rb_r * MB, MB), pl.ds(col_r, NC)],
                    store_sems.at[0])
                st1 = pltpu.make_async_copy(
                    sbuf_l, y_ref.at[pl.ds(rb_l * MB, MB), pl.ds(col_l, NC)],
                    store_sems.at[1])
                st0.start()
                st1.start()
                amax_ref[0, 0] = jnp.maximum(
                    amax_ref[0, 0],
                    jnp.maximum(
                        jnp.max(sbuf_r[...].astype(jnp.float32)),
                        jnp.max(sbuf_l[...].astype(jnp.float32))))
                st0.wait()
                st1.wait()

    pl.semaphore_wait(credit_r, 2)
    pl.semaphore_wait(credit_l, 2)


def _all_reduce(partial):
    return pl.pallas_call(
        _ar_body,
        out_shape=(
            jax.ShapeDtypeStruct((M, N), jnp.bfloat16),
            jax.ShapeDtypeStruct((1, 1), jnp.float32),
        ),
        in_specs=[pl.BlockSpec(memory_space=pl.ANY)],
        out_specs=(
            pl.BlockSpec(memory_space=pl.ANY),
            pl.BlockSpec(memory_space=pltpu.MemorySpace.SMEM),
        ),
        scratch_shapes=[
            pltpu.VMEM((MB, NC), jnp.bfloat16),
            pltpu.VMEM((MB, NC), jnp.bfloat16),
            pltpu.VMEM((MB, NC), jnp.bfloat16),
            pltpu.VMEM((MB, NC), jnp.bfloat16),
            pltpu.VMEM((2, MB, NC), jnp.bfloat16),
            pltpu.VMEM((2, MB, NC), jnp.bfloat16),
            pltpu.SemaphoreType.DMA((2,)),
            pltpu.SemaphoreType.DMA((2,)),
            pltpu.SemaphoreType.DMA((2,)),
            pltpu.SemaphoreType.DMA((2,)),
            pltpu.SemaphoreType.DMA((2,)),
            pltpu.SemaphoreType.DMA((2,)),
            pltpu.SemaphoreType.REGULAR,
            pltpu.SemaphoreType.REGULAR,
        ],
        compiler_params=pltpu.CompilerParams(collective_id=0),
    )(partial)


def _epilogue(y, amax):
    TM = 256

    def body(amax_ref, y_ref, o_ref):
        yv = jnp.maximum(y_ref[...].astype(jnp.float32), 0.0)
        scale = amax_ref[0, 0] / 127.0
        q = jnp.clip(jnp.round(yv / scale), 0.0, 127.0)
        o_ref[...] = q * scale

    return pl.pallas_call(
        body,
        grid=(M // TM,),
        out_shape=jax.ShapeDtypeStruct((M, N), jnp.float32),
        in_specs=[
            pl.BlockSpec(memory_space=pltpu.MemorySpace.SMEM),
            pl.BlockSpec((TM, N), lambda i: (i, 0)),
        ],
        out_specs=pl.BlockSpec((TM, N), lambda i: (i, 0)),
    )(amax, y)


def kernel(x, w_mat):
    xb = x.astype(jnp.bfloat16)
    wb = w_mat.astype(jnp.bfloat16)
    partial = jnp.dot(
        xb, wb, preferred_element_type=jnp.float32).astype(jnp.bfloat16)
    y, amax = _all_reduce(partial)
    return _epilogue(y, amax)


# device time: 706719 ns/iter; 1.9926x vs baseline; 1.1205x over previous
import jax
import jax.numpy as jnp
from jax import lax
from jax.experimental import pallas as pl
from jax.experimental.pallas import tpu as pltpu

N_DEV = 4
M, K, N = 4096, 1024, 8192
MB = M // N_DEV
NC = 2048
N_CHUNKS = N // NC
N_HOPS = 2 * (N_DEV - 1)


def _gemm_ar_body(x_ref, w_ref, y_ref, amax_ref,
                  sbuf_r, sbuf_l, comm_r, comm_l,
                  store_sems,
                  send_r, recv_r, send_l, recv_l,
                  credit_r, credit_l):
    me = lax.axis_index("i")
    left = lax.rem(me + N_DEV - 1, N_DEV)
    right = lax.rem(me + 1, N_DEV)

    barrier = pltpu.get_barrier_semaphore()
    for nbr in (left, right):
        pl.semaphore_signal(barrier, inc=1, device_id=(nbr,),
                            device_id_type=pl.DeviceIdType.MESH)
    pl.semaphore_wait(barrier, 2)

    amax_ref[0, 0] = 0.0

    def tile(b, col):
        return jnp.dot(
            x_ref[pl.ds(b * MB, MB), :], w_ref[:, pl.ds(col, NC)],
            preferred_element_type=jnp.float32).astype(jnp.bfloat16)

    def flush(pending):
        for st, buf in pending:
            amax_ref[0, 0] = jnp.maximum(
                amax_ref[0, 0], jnp.max(buf[...].astype(jnp.float32)))
            st.wait()

    pending = []
    n_pairs = N_CHUNKS // 2
    for p in range(n_pairs):
        col_r = p * NC
        col_l = (p + n_pairs) * NC
        flush(pending)
        pending = []
        sbuf_r[...] = tile(me, col_r)
        sbuf_l[...] = tile(me, col_l)
        for h in range(N_HOPS):
            slot = h % 2
            if p * N_HOPS + h >= 2:
                pl.semaphore_wait(credit_r, 1)
                pl.semaphore_wait(credit_l, 1)
            rd_r = pltpu.make_async_remote_copy(
                src_ref=sbuf_r, dst_ref=comm_r.at[slot],
                send_sem=send_r.at[slot], recv_sem=recv_r.at[slot],
                device_id=(right,), device_id_type=pl.DeviceIdType.MESH)
            rd_l = pltpu.make_async_remote_copy(
                src_ref=sbuf_l, dst_ref=comm_l.at[slot],
                send_sem=send_l.at[slot], recv_sem=recv_l.at[slot],
                device_id=(left,), device_id_type=pl.DeviceIdType.MESH)
            rd_r.start()
            rd_l.start()
            flush(pending)
            pending = []
            rb_r = lax.rem(me + 7 - h, N_DEV)
            rb_l = lax.rem(me + h + 1, N_DEV)
            if h < N_DEV - 1:
                pt_r = tile(rb_r, col_r)
                pt_l = tile(rb_l, col_l)
            rd_r.wait()
            rd_l.wait()
            if h < N_DEV - 1:
                sbuf_r[...] = comm_r[slot] + pt_r
                sbuf_l[...] = comm_l[slot] + pt_l
            else:
                sbuf_r[...] = comm_r[slot]
                sbuf_l[...] = comm_l[slot]
            pl.semaphore_signal(credit_r, inc=1, device_id=(left,),
                                device_id_type=pl.DeviceIdType.MESH)
            pl.semaphore_signal(credit_l, inc=1, device_id=(right,),
                                device_id_type=pl.DeviceIdType.MESH)
            if h >= N_DEV - 2:
                st0 = pltpu.make_async_copy(
                    sbuf_r, y_ref.at[pl.ds(rb_r * MB, MB), pl.ds(col_r, NC)],
                    store_sems.at[0])
                st1 = pltpu.make_async_copy(
                    sbuf_l, y_ref.at[pl.ds(rb_l * MB, MB), pl.ds(col_l, NC)],
                    store_sems.at[1])
                st0.start()
                st1.start()
                pending = [(st0, sbuf_r), (st1, sbuf_l)]

    flush(pending)
    pl.semaphore_wait(credit_r, 2)
    pl.semaphore_wait(credit_l, 2)


def _gemm_ar(xb, wb):
    return pl.pallas_call(
        _gemm_ar_body,
        out_shape=(
            jax.ShapeDtypeStruct((M, N), jnp.bfloat16),
            jax.ShapeDtypeStruct((1, 1), jnp.float32),
        ),
        in_specs=[
            pl.BlockSpec(memory_space=pltpu.MemorySpace.VMEM),
            pl.BlockSpec(memory_space=pltpu.MemorySpace.VMEM),
        ],
        out_specs=(
            pl.BlockSpec(memory_space=pl.ANY),
            pl.BlockSpec(memory_space=pltpu.MemorySpace.SMEM),
        ),
        scratch_shapes=[
            pltpu.VMEM((MB, NC), jnp.bfloat16),
            pltpu.VMEM((MB, NC), jnp.bfloat16),
            pltpu.VMEM((2, MB, NC), jnp.bfloat16),
            pltpu.VMEM((2, MB, NC), jnp.bfloat16),
            pltpu.SemaphoreType.DMA((2,)),
            pltpu.SemaphoreType.DMA((2,)),
            pltpu.SemaphoreType.DMA((2,)),
            pltpu.SemaphoreType.DMA((2,)),
            pltpu.SemaphoreType.DMA((2,)),
            pltpu.SemaphoreType.REGULAR,
            pltpu.SemaphoreType.REGULAR,
        ],
        compiler_params=pltpu.CompilerParams(
            collective_id=0, vmem_limit_bytes=64 * 1024 * 1024),
    )(xb, wb)


def _epilogue(y, amax):
    TM = 256

    def body(amax_ref, y_ref, o_ref):
        yv = jnp.maximum(y_ref[...].astype(jnp.float32), 0.0)
        scale = amax_ref[0, 0] / 127.0
        q = jnp.clip(jnp.round(yv / scale), 0.0, 127.0)
        o_ref[...] = q * scale

    return pl.pallas_call(
        body,
        grid=(M // TM,),
        out_shape=jax.ShapeDtypeStruct((M, N), jnp.float32),
        in_specs=[
            pl.BlockSpec(memory_space=pltpu.MemorySpace.SMEM),
            pl.BlockSpec((TM, N), lambda i: (i, 0)),
        ],
        out_specs=pl.BlockSpec((TM, N), lambda i: (i, 0)),
    )(amax, y)


def kernel(x, w_mat):
    xb = x.astype(jnp.bfloat16)
    wb = w_mat.astype(jnp.bfloat16)
    y, amax = _gemm_ar(xb, wb)
    return _epilogue(y, amax)


# device time: 658857 ns/iter; 2.1373x vs baseline; 1.0726x over previous
import jax
import jax.numpy as jnp
from jax import lax
from jax.experimental import pallas as pl
from jax.experimental.pallas import tpu as pltpu

N_DEV = 4
M, K, N = 4096, 1024, 8192
MB = M // N_DEV
NC = 1024
N_STREAMS = 4
N_Q = 2
N_HOPS = 2 * (N_DEV - 1)
N_ROUNDS = N_Q * N_HOPS

_BASES = (0, NC, N // 2, N // 2 + NC)
_GROUPS = ((0, 2), (1, 3))


def _col(k, q):
    return _BASES[k] + q * 2 * NC


def _gemm_ar_body(x_ref, w_ref, y_ref, amax_ref,
                  sbufs, comms, store_sems, send_sems, recv_sems,
                  credit_sems):
    me = lax.axis_index("i")
    left = lax.rem(me + N_DEV - 1, N_DEV)
    right = lax.rem(me + 1, N_DEV)

    barrier = pltpu.get_barrier_semaphore()
    for nbr in (left, right):
        pl.semaphore_signal(barrier, inc=1, device_id=(nbr,),
                            device_id_type=pl.DeviceIdType.MESH)
    pl.semaphore_wait(barrier, 2)

    amax_ref[0, 0] = 0.0

    def tile(b, col):
        return jnp.dot(
            x_ref[pl.ds(b * MB, MB), :], w_ref[:, pl.ds(col, NC)],
            preferred_element_type=jnp.float32).astype(jnp.bfloat16)

    def dst(k):
        return right if k < 2 else left

    def ups(k):
        return left if k < 2 else right

    def rb(k, h):
        if k < 2:
            return lax.rem(me + 7 - h, N_DEV)
        return lax.rem(me + h + 1, N_DEV)

    def start_send(k, g):
        slot = g % 2
        if g >= 2:
            pl.semaphore_wait(credit_sems.at[k], 1)
        r = pltpu.make_async_remote_copy(
            src_ref=sbufs.at[k], dst_ref=comms.at[k, slot],
            send_sem=send_sems.at[k, slot], recv_sem=recv_sems.at[k, slot],
            device_id=(dst(k),), device_id_type=pl.DeviceIdType.MESH)
        r.start()
        return r

    def flush(pend, k):
        if pend is None:
            return
        st, buf, needs_credit = pend
        amax_ref[0, 0] = jnp.maximum(
            amax_ref[0, 0], jnp.max(buf[...].astype(jnp.float32)))
        st.wait()
        if needs_credit:
            pl.semaphore_signal(credit_sems.at[k], inc=1, device_id=(ups(k),),
                                device_id_type=pl.DeviceIdType.MESH)

    rdma = {}
    pending = {k: None for k in range(N_STREAMS)}

    for grp in _GROUPS:
        for k in grp:
            sbufs[k, :, :] = tile(me, _col(k, 0))
        for k in grp:
            rdma[k] = start_send(k, 0)

    for r in range(N_ROUNDS):
        q, h = divmod(r, N_HOPS)
        slot = r % 2
        for grp in _GROUPS:
            for k in grp:
                if h < N_DEV - 1:
                    pt = tile(rb(k, h), _col(k, q))
                rdma[k].wait()
                flush(pending[k], k)
                pending[k] = None
                if h < N_DEV - 1:
                    sbufs[k, :, :] = comms[k, slot] + pt
                elif h < N_HOPS - 1:
                    sbufs[k, :, :] = comms[k, slot]
                if h < N_HOPS - 1:
                    pl.semaphore_signal(
                        credit_sems.at[k], inc=1, device_id=(ups(k),),
                        device_id_type=pl.DeviceIdType.MESH)
                if h >= N_DEV - 2:
                    src = sbufs.at[k] if h < N_HOPS - 1 else comms.at[k, slot]
                    st = pltpu.make_async_copy(
                        src,
                        y_ref.at[pl.ds(rb(k, h) * MB, MB),
                                 pl.ds(_col(k, q), NC)],
                        store_sems.at[k])
                    st.start()
                    pending[k] = (st, src, h == N_HOPS - 1)
                if r < N_ROUNDS - 1:
                    g2 = r + 1
                    if g2 % N_HOPS == 0:
                        sbufs[k, :, :] = tile(me, _col(k, g2 // N_HOPS))
                    rdma[k] = start_send(k, g2)

    for k in range(N_STREAMS):
        flush(pending[k], k)
    for k in range(N_STREAMS):
        pl.semaphore_wait(credit_sems.at[k], 2)


def _gemm_ar(xb, wb):
    return pl.pallas_call(
        _gemm_ar_body,
        out_shape=(
            jax.ShapeDtypeStruct((M, N), jnp.bfloat16),
            jax.ShapeDtypeStruct((1, 1), jnp.float32),
        ),
        in_specs=[
            pl.BlockSpec(memory_space=pltpu.MemorySpace.VMEM),
            pl.BlockSpec(memory_space=pltpu.MemorySpace.VMEM),
        ],
        out_specs=(
            pl.BlockSpec(memory_space=pl.ANY),
            pl.BlockSpec(memory_space=pltpu.MemorySpace.SMEM),
        ),
        scratch_shapes=[
            pltpu.VMEM((N_STREAMS, MB, NC), jnp.bfloat16),
            pltpu.VMEM((N_STREAMS, 2, MB, NC), jnp.bfloat16),
            pltpu.SemaphoreType.DMA((N_STREAMS,)),
            pltpu.SemaphoreType.DMA((N_STREAMS, 2)),
            pltpu.SemaphoreType.DMA((N_STREAMS, 2)),
            pltpu.SemaphoreType.REGULAR((N_STREAMS,)),
        ],
        compiler_params=pltpu.CompilerParams(
            collective_id=0, vmem_limit_bytes=64 * 1024 * 1024),
    )(xb, wb)


def _epilogue(y, amax):
    TM = 256

    def body(amax_ref, y_ref, o_ref):
        yv = jnp.maximum(y_ref[...].astype(jnp.float32), 0.0)
        scale = amax_ref[0, 0] / 127.0
        q = jnp.clip(jnp.round(yv / scale), 0.0, 127.0)
        o_ref[...] = q * scale

    return pl.pallas_call(
        body,
        grid=(M // TM,),
        out_shape=jax.ShapeDtypeStruct((M, N), jnp.float32),
        in_specs=[
            pl.BlockSpec(memory_space=pltpu.MemorySpace.SMEM),
            pl.BlockSpec((TM, N), lambda i: (i, 0)),
        ],
        out_specs=pl.BlockSpec((TM, N), lambda i: (i, 0)),
    )(amax, y)


def kernel(x, w_mat):
    xb = x.astype(jnp.bfloat16)
    wb = w_mat.astype(jnp.bfloat16)
    y, amax = _gemm_ar(xb, wb)
    return _epilogue(y, amax)


# device time: 526958 ns/iter; 2.6723x vs baseline; 1.2503x over previous
import jax
import jax.numpy as jnp
from jax import lax
from jax.experimental import pallas as pl
from jax.experimental.pallas import tpu as pltpu

N_DEV = 4
M, K, N = 4096, 1024, 8192
MB = M // N_DEV
NC = 1024
N_STREAMS = 4
N_Q = 2
N_RS = N_DEV - 1
RS_ROUNDS = N_Q * N_RS
AG_ROUNDS = N_Q * N_RS

_BASES = (0, NC, N // 2, N // 2 + NC)
_GROUPS = ((0, 2), (1, 3))


def _col(k, q):
    return _BASES[k] + q * 2 * NC


def _gemm_ar_body(x_ref, w_ref, yq_ref, stage_ref, amax_ref,
                  sbufs, sbufs_i8, comms, comm_i8, abuf,
                  store_sems, stage_sems, send_bf, recv_bf,
                  send_i8, recv_i8, a_send, a_recv,
                  credit_bf, credit_i8):
    me = lax.axis_index("i")
    left = lax.rem(me + N_DEV - 1, N_DEV)
    right = lax.rem(me + 1, N_DEV)

    barrier = pltpu.get_barrier_semaphore()
    for nbr in (left, right):
        pl.semaphore_signal(barrier, inc=1, device_id=(nbr,),
                            device_id_type=pl.DeviceIdType.MESH)
    pl.semaphore_wait(barrier, 2)

    amax_ref[0, 0] = 0.0

    def tile(b, col):
        return jnp.dot(
            x_ref[pl.ds(b * MB, MB), :], w_ref[:, pl.ds(col, NC)],
            preferred_element_type=jnp.float32).astype(jnp.bfloat16)

    def dst(k):
        return right if k < 2 else left

    def ups(k):
        return left if k < 2 else right

    def rb(k, h):
        if k < 2:
            return lax.rem(me + 7 - h, N_DEV)
        return lax.rem(me + h + 1, N_DEV)

    def own(k):
        return rb(k, 2)

    def quantize(v):
        yv = jnp.maximum(v.astype(jnp.float32), 0.0)
        scale = amax_ref[0, 0] / 127.0
        return jnp.clip(jnp.round(yv / scale), 0.0, 127.0).astype(jnp.int8)

    pending = {k: [] for k in range(N_STREAMS)}
    store_slot = [0] * N_STREAMS

    def flush(k):
        for st, buf, credit in pending[k]:
            if buf is not None:
                amax_ref[0, 0] = jnp.maximum(
                    amax_ref[0, 0], jnp.max(buf[...].astype(jnp.float32)))
            st.wait()
            if credit is not None:
                pl.semaphore_signal(credit.at[k], inc=1, device_id=(ups(k),),
                                    device_id_type=pl.DeviceIdType.MESH)
        pending[k] = []

    def store_q(k, src, block, q):
        s = store_slot[k]
        store_slot[k] = 1 - s
        st = pltpu.make_async_copy(
            src, yq_ref.at[pl.ds(block * MB, MB), pl.ds(_col(k, q), NC)],
            store_sems.at[k, s])
        st.start()
        return st

    def start_send_bf(k, g):
        slot = g % 2
        if g >= 2:
            pl.semaphore_wait(credit_bf.at[k], 1)
        r = pltpu.make_async_remote_copy(
            src_ref=sbufs.at[k], dst_ref=comms.at[k, slot],
            send_sem=send_bf.at[k, slot], recv_sem=recv_bf.at[k, slot],
            device_id=(dst(k),), device_id_type=pl.DeviceIdType.MESH)
        r.start()
        return r

    rdma = {}
    for grp in _GROUPS:
        for k in grp:
            sbufs[k, :, :] = tile(me, _col(k, 0))
        for k in grp:
            rdma[k] = start_send_bf(k, 0)

    for r in range(RS_ROUNDS):
        q, h = divmod(r, N_RS)
        slot = r % 2
        for grp in _GROUPS:
            for k in grp:
                pt = tile(rb(k, h), _col(k, q))
                rdma[k].wait()
                flush(k)
                sbufs[k, :, :] = comms[k, slot] + pt
                pl.semaphore_signal(
                    credit_bf.at[k], inc=1, device_id=(ups(k),),
                    device_id_type=pl.DeviceIdType.MESH)
                if h == N_RS - 1:
                    st = pltpu.make_async_copy(
                        sbufs.at[k], stage_ref.at[2 * k + q],
                        stage_sems.at[k])
                    st.start()
                    pending[k].append((st, sbufs.at[k], None))
                if r < RS_ROUNDS - 1:
                    g2 = r + 1
                    if g2 % N_RS == 0:
                        flush(k)
                        sbufs[k, :, :] = tile(me, _col(k, g2 // N_RS))
                    rdma[k] = start_send_bf(k, g2)

    for k in range(N_STREAMS):
        flush(k)

    abuf[0, :, :] = jnp.full((8, 128), amax_ref[0, 0], jnp.float32)
    for hop in range(N_DEV - 1):
        ar = pltpu.make_async_remote_copy(
            src_ref=abuf.at[0], dst_ref=abuf.at[1],
            send_sem=a_send.at[hop], recv_sem=a_recv.at[hop],
            device_id=(right,), device_id_type=pl.DeviceIdType.MESH)
        ar.start()
        ar.wait()
        abuf[0, :, :] = jnp.maximum(abuf[0], abuf[1])
    amax_ref[0, 0] = abuf[0, 0, 0]

    def start_send_i8(k, a):
        if a >= 1:
            pl.semaphore_wait(credit_i8.at[k], 1)
        r = pltpu.make_async_remote_copy(
            src_ref=sbufs_i8.at[k], dst_ref=comm_i8.at[k],
            send_sem=send_i8.at[k], recv_sem=recv_i8.at[k],
            device_id=(dst(k),), device_id_type=pl.DeviceIdType.MESH)
        r.start()
        return r

    rbk = {}
    for grp in _GROUPS:
        for k in grp:
            rd = pltpu.make_async_copy(
                stage_ref.at[2 * k], sbufs.at[k], stage_sems.at[k])
            rd.start()
            rd.wait()
            sbufs_i8[k, :, :] = quantize(sbufs[k])
            pending[k].append((store_q(k, sbufs_i8.at[k], own(k), 0),
                               None, None))
            rbk[k] = pltpu.make_async_copy(
                stage_ref.at[2 * k + 1], sbufs.at[k], stage_sems.at[k])
            rbk[k].start()
        for k in grp:
            rdma[k] = start_send_i8(k, 0)

    for a in range(AG_ROUNDS):
        q, hh = divmod(a, N_RS)
        for grp in _GROUPS:
            for k in grp:
                rdma[k].wait()
                flush(k)
                blk = rb(k, 3 + hh)
                sbufs_i8[k, :, :] = comm_i8[k]
                pl.semaphore_signal(
                    credit_i8.at[k], inc=1, device_id=(ups(k),),
                    device_id_type=pl.DeviceIdType.MESH)
                pending[k].append(
                    (store_q(k, sbufs_i8.at[k], blk, q), None, None))
                if a < AG_ROUNDS - 1:
                    a2 = a + 1
                    if a2 % N_RS == 0:
                        flush(k)
                        rbk[k].wait()
                        sbufs_i8[k, :, :] = quantize(sbufs[k])
                        pending[k].append(
                            (store_q(k, sbufs_i8.at[k], own(k), 1),
                             None, None))
                    rdma[k] = start_send_i8(k, a2)

    for k in range(N_STREAMS):
        flush(k)
    for k in range(N_STREAMS):
        pl.semaphore_wait(credit_bf.at[k], 2)
        pl.semaphore_wait(credit_i8.at[k], 1)


def _gemm_ar(xb, wb):
    return pl.pallas_call(
        _gemm_ar_body,
        out_shape=(
            jax.ShapeDtypeStruct((M, N), jnp.int8),
            jax.ShapeDtypeStruct((2 * N_STREAMS, MB, NC), jnp.bfloat16),
            jax.ShapeDtypeStruct((1, 1), jnp.float32),
        ),
        in_specs=[
            pl.BlockSpec(memory_space=pltpu.MemorySpace.VMEM),
            pl.BlockSpec(memory_space=pltpu.MemorySpace.VMEM),
        ],
        out_specs=(
            pl.BlockSpec(memory_space=pl.ANY),
            pl.BlockSpec(memory_space=pl.ANY),
            pl.BlockSpec(memory_space=pltpu.MemorySpace.SMEM),
        ),
        scratch_shapes=[
            pltpu.VMEM((N_STREAMS, MB, NC), jnp.bfloat16),
            pltpu.VMEM((N_STREAMS, MB, NC), jnp.int8),
            pltpu.VMEM((N_STREAMS, 2, MB, NC), jnp.bfloat16),
            pltpu.VMEM((N_STREAMS, MB, NC), jnp.int8),
            pltpu.VMEM((2, 8, 128), jnp.float32),
            pltpu.SemaphoreType.DMA((N_STREAMS, 2)),
            pltpu.SemaphoreType.DMA((N_STREAMS,)),
            pltpu.SemaphoreType.DMA((N_STREAMS, 2)),
            pltpu.SemaphoreType.DMA((N_STREAMS, 2)),
            pltpu.SemaphoreType.DMA((N_STREAMS,)),
            pltpu.SemaphoreType.DMA((N_STREAMS,)),
            pltpu.SemaphoreType.DMA((N_DEV - 1,)),
            pltpu.SemaphoreType.DMA((N_DEV - 1,)),
            pltpu.SemaphoreType.REGULAR((N_STREAMS,)),
            pltpu.SemaphoreType.REGULAR((N_STREAMS,)),
        ],
        compiler_params=pltpu.CompilerParams(
            collective_id=0, vmem_limit_bytes=64 * 1024 * 1024),
    )(xb, wb)


def _epilogue(yq, amax):
    TM = 256

    def body(amax_ref, q_ref, o_ref):
        scale = amax_ref[0, 0] / 127.0
        o_ref[...] = q_ref[...].astype(jnp.float32) * scale

    return pl.pallas_call(
        body,
        grid=(M // TM,),
        out_shape=jax.ShapeDtypeStruct((M, N), jnp.float32),
        in_specs=[
            pl.BlockSpec(memory_space=pltpu.MemorySpace.SMEM),
            pl.BlockSpec((TM, N), lambda i: (i, 0)),
        ],
        out_specs=pl.BlockSpec((TM, N), lambda i: (i, 0)),
    )(amax, yq)


def kernel(x, w_mat):
    xb = x.astype(jnp.bfloat16)
    wb = w_mat.astype(jnp.bfloat16)
    yq, _stage, amax = _gemm_ar(xb, wb)
    return _epilogue(yq, amax)
